# baseline (speedup 1.0000x reference)
import jax
import jax.numpy as jnp
from jax import lax
from jax.experimental import pallas as pl
from jax.experimental.pallas import tpu as pltpu

_NBINS = 10
_POOL = 8
_H = 512
_W = 512
_V = 4


def _hog_kernel(x_ref, out_ref):
    vh = _V * _H
    img = x_ref[...].reshape(vh, _W).astype(jnp.bfloat16).astype(jnp.float32)

    rows = lax.broadcasted_iota(jnp.int32, (vh, _W), 0)
    cols = lax.broadcasted_iota(jnp.int32, (vh, _W), 1)
    rmod = rows & (_H - 1)
    zero = jnp.zeros_like(img)

    rm1 = jnp.where(rmod == 0, zero, pltpu.roll(img, 1, 0))
    rp1 = jnp.where(rmod == _H - 1, zero, pltpu.roll(img, vh - 1, 0))
    s = rm1 + 2.0 * img + rp1
    d = rm1 - rp1
    scm = jnp.where(cols == 0, zero, pltpu.roll(s, 1, 1))
    scp = jnp.where(cols == _W - 1, zero, pltpu.roll(s, _W - 1, 1))
    dcm = jnp.where(cols == 0, zero, pltpu.roll(d, 1, 1))
    dcp = jnp.where(cols == _W - 1, zero, pltpu.roll(d, _W - 1, 1))
    gx = scm - scp
    gy = dcm + 2.0 * d + dcp

    mag = jnp.sqrt(gx * gx + gy * gy)

    ax = jnp.abs(gx)
    ay = jnp.abs(gy)
    mx = jnp.maximum(ax, ay)
    mn = jnp.minimum(ax, ay)
    mx2 = mx * mx
    t = mn * lax.rsqrt(jnp.where(mx2 == 0.0, jnp.float32(1.0), mx2))
    t2 = t * t
    acc = jnp.float32(0.02303680912309386)
    for c in (-0.1110499768581103, 0.2581147357390266, -0.4237426564279158,
              0.6311467942534973, -1.0605939155225785, 3.183088541784539):
        acc = acc * t2 + jnp.float32(c)
    f = acc * t
    a0 = jnp.where(ax > ay, 5.0 - f, f)
    a1 = jnp.where(gy < 0.0, 10.0 - a0, a0)
    phase_int = jnp.where(gx < 0.0, -a1, a1)
    ffloor = jnp.floor(phase_int)
    int_case = ffloor == phase_int
    fbin = jnp.where(ffloor < 0.0, ffloor + 10.0, ffloor)
    fbin = jnp.where(fbin >= 10.0, fbin - 10.0, fbin)

    a_val = jnp.where(int_case, jnp.float32(1.0), mag)
    b_val = 1.0 - a_val

    pcol_w = lax.broadcasted_iota(jnp.int32, (_W, _W // _POOL), 1)
    prow_w = lax.broadcasted_iota(jnp.int32, (_W, _W // _POOL), 0)
    pmat_w = jnp.where(prow_w // _POOL == pcol_w, 1.0 / _POOL, 0.0).astype(
        jnp.float32
    )
    prow_h = lax.broadcasted_iota(jnp.int32, (vh, vh // _POOL), 0)
    pcol_h = lax.broadcasted_iota(jnp.int32, (vh, vh // _POOL), 1)
    pmat_h = jnp.where(prow_h // _POOL == pcol_h, 1.0 / _POOL, 0.0).astype(
        jnp.float32
    )

    hp = _H // _POOL
    masks = [fbin == jnp.float32(b) for b in range(_NBINS)]
    for b in range(_NBINS):
        contrib = jnp.where(
            masks[b], a_val, jnp.where(masks[b - 1], b_val, zero)
        )
        tmp = jnp.dot(contrib, pmat_w, preferred_element_type=jnp.float32)
        pooled = lax.dot_general(
            pmat_h, tmp, (((0,), (0,)), ((), ())),
            preferred_element_type=jnp.float32,
        )
        for v in range(_V):
            out_ref[v, b] = pooled[v * hp:(v + 1) * hp, :]


@jax.jit
def kernel(x, weight):
    del weight
    n = x.shape[0]
    return pl.pallas_call(
        _hog_kernel,
        grid=(n // _V,),
        in_specs=[
            pl.BlockSpec((_V, 1, _H, _W), lambda i: (i, 0, 0, 0)),
        ],
        out_specs=pl.BlockSpec(
            (_V, _NBINS, _H // _POOL, _W // _POOL), lambda i: (i, 0, 0, 0)
        ),
        out_shape=jax.ShapeDtypeStruct(
            (n, _NBINS, _H // _POOL, _W // _POOL), jnp.float32
        ),
        compiler_params=pltpu.CompilerParams(
            dimension_semantics=(pltpu.PARALLEL,),
        ),
    )(x)


# speedup vs baseline: 1.0084x; 1.0084x over previous
import jax
import jax.numpy as jnp
from jax import lax
from jax.experimental import pallas as pl
from jax.experimental.pallas import tpu as pltpu

_NBINS = 10
_POOL = 8
_H = 512
_W = 512
_V = 4


def _hog_kernel(x_ref, out_ref):
    vh = _V * _H
    img = x_ref[...].reshape(vh, _W).astype(jnp.bfloat16).astype(jnp.float32)

    rows = lax.broadcasted_iota(jnp.int32, (vh, _W), 0)
    cols = lax.broadcasted_iota(jnp.int32, (vh, _W), 1)
    rmod = rows & (_H - 1)
    zero = jnp.zeros_like(img)

    rm1 = jnp.where(rmod == 0, zero, pltpu.roll(img, 1, 0))
    rp1 = jnp.where(rmod == _H - 1, zero, pltpu.roll(img, vh - 1, 0))
    s = rm1 + 2.0 * img + rp1
    d = rm1 - rp1
    scm = jnp.where(cols == 0, zero, pltpu.roll(s, 1, 1))
    scp = jnp.where(cols == _W - 1, zero, pltpu.roll(s, _W - 1, 1))
    dcm = jnp.where(cols == 0, zero, pltpu.roll(d, 1, 1))
    dcp = jnp.where(cols == _W - 1, zero, pltpu.roll(d, _W - 1, 1))
    gx = scm - scp
    gy = dcm + 2.0 * d + dcp

    mag = jnp.sqrt(gx * gx + gy * gy)

    ax = jnp.abs(gx)
    ay = jnp.abs(gy)
    mx = jnp.maximum(ax, ay)
    mn = jnp.minimum(ax, ay)
    t = mn / jnp.where(mx == 0.0, jnp.float32(1.0), mx)
    t2 = t * t
    acc = jnp.float32(0.02303680912309386)
    for c in (-0.1110499768581103, 0.2581147357390266, -0.4237426564279158,
              0.6311467942534973, -1.0605939155225785, 3.183088541784539):
        acc = acc * t2 + jnp.float32(c)
    f = acc * t
    a0 = jnp.where(ax > ay, 5.0 - f, f)
    a1 = jnp.where(gy < 0.0, 10.0 - a0, a0)
    phase_int = jnp.where(gx < 0.0, -a1, a1)
    ffloor = jnp.floor(phase_int)
    int_case = ffloor == phase_int
    fbin = jnp.where(ffloor < 0.0, ffloor + 10.0, ffloor)
    fbin = jnp.where(fbin >= 10.0, fbin - 10.0, fbin)

    a_val = jnp.where(int_case, jnp.float32(1.0), mag)
    b_val = 1.0 - a_val

    pcol_w = lax.broadcasted_iota(jnp.int32, (_W, _W // _POOL), 1)
    prow_w = lax.broadcasted_iota(jnp.int32, (_W, _W // _POOL), 0)
    pmat_w = jnp.where(prow_w // _POOL == pcol_w, 1.0 / _POOL, 0.0).astype(
        jnp.float32
    )
    prow_h = lax.broadcasted_iota(jnp.int32, (vh, vh // _POOL), 0)
    pcol_h = lax.broadcasted_iota(jnp.int32, (vh, vh // _POOL), 1)
    pmat_h = jnp.where(prow_h // _POOL == pcol_h, 1.0 / _POOL, 0.0).astype(
        jnp.float32
    )

    hp = _H // _POOL
    masks = [fbin == jnp.float32(b) for b in range(_NBINS)]
    for b in range(_NBINS):
        contrib = jnp.where(
            masks[b], a_val, jnp.where(masks[b - 1], b_val, zero)
        )
        tmp = jnp.dot(contrib, pmat_w, preferred_element_type=jnp.float32)
        pooled = lax.dot_general(
            pmat_h, tmp, (((0,), (0,)), ((), ())),
            preferred_element_type=jnp.float32,
        )
        for v in range(_V):
            out_ref[v, b] = pooled[v * hp:(v + 1) * hp, :]


@jax.jit
def kernel(x, weight):
    del weight
    n = x.shape[0]
    return pl.pallas_call(
        _hog_kernel,
        grid=(n // _V,),
        in_specs=[
            pl.BlockSpec((_V, 1, _H, _W), lambda i: (i, 0, 0, 0)),
        ],
        out_specs=pl.BlockSpec(
            (_V, _NBINS, _H // _POOL, _W // _POOL), lambda i: (i, 0, 0, 0)
        ),
        out_shape=jax.ShapeDtypeStruct(
            (n, _NBINS, _H // _POOL, _W // _POOL), jnp.float32
        ),
        compiler_params=pltpu.CompilerParams(
            dimension_semantics=(pltpu.PARALLEL,),
        ),
    )(x)
